# baseline (speedup 1.0000x reference)
import functools

import jax
import jax.numpy as jnp
from jax.experimental import pallas as pl
from jax.experimental.pallas import tpu as pltpu

_TOPK = 10
_KCAP = 16


def _fused_topk_kernel(nkeys, bk, nblk,
                       q_ref, nq_ref, kb_ref, nk_ref,
                       vals_ref, idx_ref,
                       s_ref, v_ref, i_ref):
    blk = pl.program_id(0)
    nq_rows = q_ref.shape[0]

    @pl.when(blk == 0)
    def _init():
        v_ref[...] = jnp.full((nq_rows, _KCAP), -jnp.inf, jnp.float32)
        i_ref[...] = jnp.zeros((nq_rows, _KCAP), jnp.int32)

    dots = jax.lax.dot_general(
        q_ref[...], kb_ref[...], (((1,), (1,)), ((), ())),
        preferred_element_type=jnp.float32)
    denom = nk_ref[...] * nq_ref[...] + 1e-8
    sims = dots / denom
    col_ids = jax.lax.broadcasted_iota(jnp.int32, (nq_rows, bk), 1)
    sims = jnp.where(col_ids + blk * bk < nkeys, sims, -jnp.inf)
    s_ref[...] = sims

    m0 = jnp.max(sims, axis=1, keepdims=True)
    cont0 = jnp.any(m0 > v_ref[:, _TOPK - 1:_TOPK])

    def cond(carry):
        return carry[0]

    def body(carry):
        _, m = carry
        s = s_ref[...]
        mi = jnp.argmax(s, axis=1)[:, None].astype(jnp.int32)
        v = v_ref[...]
        iv = i_ref[...]
        tmin = v[:, _TOPK - 1:_TOPK]
        upd = m > tmin
        cols = jax.lax.broadcasted_iota(jnp.int32, (nq_rows, bk), 1)
        s_next = jnp.where(cols == mi, -jnp.inf, s)
        s_ref[...] = s_next
        pos = jnp.sum((v >= m).astype(jnp.int32), axis=1, keepdims=True)
        kcols = jax.lax.broadcasted_iota(jnp.int32, (nq_rows, _KCAP), 1)
        v_shift = jnp.concatenate([v[:, :1], v[:, :_KCAP - 1]], axis=1)
        i_shift = jnp.concatenate([iv[:, :1], iv[:, :_KCAP - 1]], axis=1)
        gidx = mi + blk * bk
        newv = jnp.where(kcols < pos, v,
                         jnp.where(kcols == pos, m, v_shift))
        newi = jnp.where(kcols < pos, iv,
                         jnp.where(kcols == pos, gidx, i_shift))
        v_ref[...] = jnp.where(upd, newv, v)
        i_ref[...] = jnp.where(upd, newi, iv)
        m2 = jnp.max(s_next, axis=1, keepdims=True)
        cont = jnp.any(m2 > v_ref[:, _TOPK - 1:_TOPK])
        return cont, m2

    jax.lax.while_loop(cond, body, (cont0, m0))

    @pl.when(blk == nblk - 1)
    def _emit():
        vals_ref[...] = v_ref[:, :_TOPK]
        idx_ref[...] = i_ref[:, :_TOPK]


def kernel(queries, keys, k):
    del k
    q, d = queries.shape
    nkeys = keys.shape[0]
    bk = 2048
    nblk = (nkeys + bk - 1) // bk
    kpad = nblk * bk

    norms_text = jnp.linalg.norm(keys, axis=-1)
    norm_question = jnp.linalg.norm(queries, axis=-1, keepdims=True)

    keys_pad = jnp.pad(keys, ((0, kpad - nkeys), (0, 0)))
    nk = jnp.pad(norms_text, (0, kpad - nkeys))[None, :]

    body = functools.partial(_fused_topk_kernel, nkeys, bk, nblk)
    vals, idx = pl.pallas_call(
        body,
        grid=(nblk,),
        in_specs=[
            pl.BlockSpec((q, d), lambda i: (0, 0)),
            pl.BlockSpec((q, 1), lambda i: (0, 0)),
            pl.BlockSpec((bk, d), lambda i: (i, 0)),
            pl.BlockSpec((1, bk), lambda i: (0, i)),
        ],
        out_specs=[
            pl.BlockSpec((q, _TOPK), lambda i: (0, 0)),
            pl.BlockSpec((q, _TOPK), lambda i: (0, 0)),
        ],
        out_shape=[
            jax.ShapeDtypeStruct((q, _TOPK), jnp.float32),
            jax.ShapeDtypeStruct((q, _TOPK), jnp.int32),
        ],
        scratch_shapes=[
            pltpu.VMEM((q, bk), jnp.float32),
            pltpu.VMEM((q, _KCAP), jnp.float32),
            pltpu.VMEM((q, _KCAP), jnp.int32),
        ],
        compiler_params=pltpu.CompilerParams(
            dimension_semantics=("arbitrary",)),
    )(queries, norm_question, keys_pad, nk)

    return vals, idx


# speedup vs baseline: 1.1806x; 1.1806x over previous
import functools

import jax
import jax.numpy as jnp
from jax.experimental import pallas as pl
from jax.experimental.pallas import tpu as pltpu

_TOPK = 10
_KCAP = 16


def _fused_topk_kernel(nkeys, bk, nblk,
                       qt_ref, nq_ref, kb_ref, nk_ref,
                       vals_ref, idx_ref,
                       s_ref, v_ref, i_ref):
    blk = pl.program_id(0)
    nq_cols = qt_ref.shape[1]

    @pl.when(blk == 0)
    def _init():
        v_ref[...] = jnp.full((_KCAP, nq_cols), -jnp.inf, jnp.float32)
        i_ref[...] = jnp.zeros((_KCAP, nq_cols), jnp.int32)

    dots = jax.lax.dot_general(
        kb_ref[...], qt_ref[...], (((1,), (0,)), ((), ())),
        preferred_element_type=jnp.float32)
    denom = nk_ref[...] * nq_ref[...] + 1e-8
    sims = dots / denom
    row_ids = jax.lax.broadcasted_iota(jnp.int32, (bk, nq_cols), 0)
    sims = jnp.where(row_ids + blk * bk < nkeys, sims, -jnp.inf)
    s_ref[...] = sims

    m0 = jnp.max(sims, axis=0, keepdims=True)
    cont0 = jnp.any(m0 > v_ref[_TOPK - 1:_TOPK, :])

    def cond(carry):
        return carry[0]

    def body(carry):
        _, m = carry
        s = s_ref[...]
        mi = jnp.argmax(s, axis=0)[None, :].astype(jnp.int32)
        v = v_ref[...]
        iv = i_ref[...]
        tmin = v[_TOPK - 1:_TOPK, :]
        upd = m > tmin
        rows = jax.lax.broadcasted_iota(jnp.int32, (bk, nq_cols), 0)
        s_next = jnp.where(rows == mi, -jnp.inf, s)
        s_ref[...] = s_next
        pos = jnp.sum((v >= m).astype(jnp.int32), axis=0, keepdims=True)
        krows = jax.lax.broadcasted_iota(jnp.int32, (_KCAP, nq_cols), 0)
        v_shift = jnp.concatenate([v[:1], v[:_KCAP - 1]], axis=0)
        i_shift = jnp.concatenate([iv[:1], iv[:_KCAP - 1]], axis=0)
        gidx = mi + blk * bk
        newv = jnp.where(krows < pos, v,
                         jnp.where(krows == pos, m, v_shift))
        newi = jnp.where(krows < pos, iv,
                         jnp.where(krows == pos, gidx, i_shift))
        v_ref[...] = jnp.where(upd, newv, v)
        i_ref[...] = jnp.where(upd, newi, iv)
        m2 = jnp.max(s_next, axis=0, keepdims=True)
        cont = jnp.any(m2 > v_ref[_TOPK - 1:_TOPK, :])
        return cont, m2

    jax.lax.while_loop(cond, body, (cont0, m0))

    @pl.when(blk == nblk - 1)
    def _emit():
        vals_ref[...] = v_ref[:_TOPK, :]
        idx_ref[...] = i_ref[:_TOPK, :]


def kernel(queries, keys, k):
    del k
    q, d = queries.shape
    nkeys = keys.shape[0]
    bk = 2048
    nblk = (nkeys + bk - 1) // bk
    kpad = nblk * bk

    norms_text = jnp.linalg.norm(keys, axis=-1)
    norm_question = jnp.linalg.norm(queries, axis=-1)

    qt = queries.T
    keys_pad = jnp.pad(keys, ((0, kpad - nkeys), (0, 0)))
    nk = jnp.pad(norms_text, (0, kpad - nkeys))[:, None]
    nq = norm_question[None, :]

    body = functools.partial(_fused_topk_kernel, nkeys, bk, nblk)
    vals_t, idx_t = pl.pallas_call(
        body,
        grid=(nblk,),
        in_specs=[
            pl.BlockSpec((d, q), lambda i: (0, 0)),
            pl.BlockSpec((1, q), lambda i: (0, 0)),
            pl.BlockSpec((bk, d), lambda i: (i, 0)),
            pl.BlockSpec((bk, 1), lambda i: (i, 0)),
        ],
        out_specs=[
            pl.BlockSpec((_TOPK, q), lambda i: (0, 0)),
            pl.BlockSpec((_TOPK, q), lambda i: (0, 0)),
        ],
        out_shape=[
            jax.ShapeDtypeStruct((_TOPK, q), jnp.float32),
            jax.ShapeDtypeStruct((_TOPK, q), jnp.int32),
        ],
        scratch_shapes=[
            pltpu.VMEM((bk, q), jnp.float32),
            pltpu.VMEM((_KCAP, q), jnp.float32),
            pltpu.VMEM((_KCAP, q), jnp.int32),
        ],
        compiler_params=pltpu.CompilerParams(
            dimension_semantics=("arbitrary",)),
    )(qt, nq, keys_pad, nk)

    return vals_t.T, idx_t.T


# speedup vs baseline: 1.1960x; 1.0131x over previous
import functools

import jax
import jax.numpy as jnp
from jax.experimental import pallas as pl
from jax.experimental.pallas import tpu as pltpu

_TOPK = 10
_KCAP = 16


def _fused_topk_kernel(nkeys, bk, nblk,
                       qt_ref, nq_ref, kb_ref, nk_ref,
                       vals_ref, idx_ref,
                       s_ref, v_ref, i_ref):
    blk = pl.program_id(0)
    nq_cols = qt_ref.shape[1]

    @pl.when(blk == 0)
    def _init():
        v_ref[...] = jnp.full((_KCAP, nq_cols), -jnp.inf, jnp.float32)
        i_ref[...] = jnp.zeros((_KCAP, nq_cols), jnp.int32)

    dots = jax.lax.dot_general(
        kb_ref[...], qt_ref[...], (((1,), (0,)), ((), ())),
        preferred_element_type=jnp.float32)
    denom = nk_ref[...] * nq_ref[...] + 1e-8
    s_ref[...] = dots / denom

    @pl.when(blk == nblk - 1)
    def _mask_pad():
        row_ids = jax.lax.broadcasted_iota(jnp.int32, (bk, nq_cols), 0)
        s_ref[...] = jnp.where(row_ids + blk * bk < nkeys,
                               s_ref[...], -jnp.inf)

    m0 = jnp.max(s_ref[...], axis=0, keepdims=True)
    cont0 = jnp.any(m0 > v_ref[_TOPK - 1:_TOPK, :])

    def cond(carry):
        return carry[0]

    def body(carry):
        _, m = carry
        s = s_ref[...]
        mi = jnp.argmax(s, axis=0)[None, :].astype(jnp.int32)
        v = v_ref[...]
        iv = i_ref[...]
        tmin = v[_TOPK - 1:_TOPK, :]
        upd = m > tmin
        rows = jax.lax.broadcasted_iota(jnp.int32, (bk, nq_cols), 0)
        s_next = jnp.where(rows == mi, -jnp.inf, s)
        s_ref[...] = s_next
        pos = jnp.sum((v >= m).astype(jnp.int32), axis=0, keepdims=True)
        krows = jax.lax.broadcasted_iota(jnp.int32, (_KCAP, nq_cols), 0)
        v_shift = jnp.concatenate([v[:1], v[:_KCAP - 1]], axis=0)
        i_shift = jnp.concatenate([iv[:1], iv[:_KCAP - 1]], axis=0)
        gidx = mi + blk * bk
        newv = jnp.where(krows < pos, v,
                         jnp.where(krows == pos, m, v_shift))
        newi = jnp.where(krows < pos, iv,
                         jnp.where(krows == pos, gidx, i_shift))
        v_ref[...] = jnp.where(upd, newv, v)
        i_ref[...] = jnp.where(upd, newi, iv)
        m2 = jnp.max(s_next, axis=0, keepdims=True)
        cont = jnp.any(m2 > v_ref[_TOPK - 1:_TOPK, :])
        return cont, m2

    jax.lax.while_loop(cond, body, (cont0, m0))

    @pl.when(blk == nblk - 1)
    def _emit():
        vals_ref[...] = v_ref[:_TOPK, :]
        idx_ref[...] = i_ref[:_TOPK, :]


def kernel(queries, keys, k):
    del k
    q, d = queries.shape
    nkeys = keys.shape[0]
    bk = 2048
    nblk = (nkeys + bk - 1) // bk
    kpad = nblk * bk

    norms_text = jnp.linalg.norm(keys, axis=-1)
    norm_question = jnp.linalg.norm(queries, axis=-1)

    qt = queries.T
    keys_pad = jnp.pad(keys, ((0, kpad - nkeys), (0, 0)))
    nk = jnp.pad(norms_text, (0, kpad - nkeys))[:, None]
    nq = norm_question[None, :]

    body = functools.partial(_fused_topk_kernel, nkeys, bk, nblk)
    vals_t, idx_t = pl.pallas_call(
        body,
        grid=(nblk,),
        in_specs=[
            pl.BlockSpec((d, q), lambda i: (0, 0)),
            pl.BlockSpec((1, q), lambda i: (0, 0)),
            pl.BlockSpec((bk, d), lambda i: (i, 0)),
            pl.BlockSpec((bk, 1), lambda i: (i, 0)),
        ],
        out_specs=[
            pl.BlockSpec((_TOPK, q), lambda i: (0, 0)),
            pl.BlockSpec((_TOPK, q), lambda i: (0, 0)),
        ],
        out_shape=[
            jax.ShapeDtypeStruct((_TOPK, q), jnp.float32),
            jax.ShapeDtypeStruct((_TOPK, q), jnp.int32),
        ],
        scratch_shapes=[
            pltpu.VMEM((bk, q), jnp.float32),
            pltpu.VMEM((_KCAP, q), jnp.float32),
            pltpu.VMEM((_KCAP, q), jnp.int32),
        ],
        compiler_params=pltpu.CompilerParams(
            dimension_semantics=("arbitrary",)),
    )(qt, nq, keys_pad, nk)

    return vals_t.T, idx_t.T


# speedup vs baseline: 1.3012x; 1.0879x over previous
import functools

import jax
import jax.numpy as jnp
from jax.experimental import pallas as pl
from jax.experimental.pallas import tpu as pltpu

_TOPK = 10
_KCAP = 16


def _fused_topk_kernel(nkeys, bk, nblk,
                       qt_ref, nq_ref, kb_ref, nk_ref,
                       vals_ref, idx_ref,
                       s_ref, v_ref, i_ref):
    blk = pl.program_id(0)
    nq_cols = qt_ref.shape[1]

    @pl.when(blk == 0)
    def _init():
        v_ref[...] = jnp.full((_KCAP, nq_cols), -jnp.inf, jnp.float32)
        i_ref[...] = jnp.zeros((_KCAP, nq_cols), jnp.int32)

    dots = jax.lax.dot_general(
        kb_ref[...], qt_ref[...], (((1,), (0,)), ((), ())),
        preferred_element_type=jnp.float32)
    denom = nk_ref[...] * nq_ref[...] + 1e-8
    s_ref[...] = dots / denom

    @pl.when(blk == nblk - 1)
    def _mask_pad():
        row_ids = jax.lax.broadcasted_iota(jnp.int32, (bk, nq_cols), 0)
        s_ref[...] = jnp.where(row_ids + blk * bk < nkeys,
                               s_ref[...], -jnp.inf)

    m0 = jnp.max(s_ref[...], axis=0, keepdims=True)
    cont0 = jnp.any(m0 > v_ref[_TOPK - 1:_TOPK, :])

    def cond(carry):
        return carry[0]

    def body(carry):
        _, m = carry
        s = s_ref[...]
        mi = jnp.argmax(s, axis=0)[None, :].astype(jnp.int32)
        v = v_ref[...]
        iv = i_ref[...]
        tmin = v[_TOPK - 1:_TOPK, :]
        upd = m > tmin
        rows = jax.lax.broadcasted_iota(jnp.int32, (bk, nq_cols), 0)
        s_next = jnp.where(rows == mi, -jnp.inf, s)
        s_ref[...] = s_next
        pos = jnp.sum((v >= m).astype(jnp.int32), axis=0, keepdims=True)
        krows = jax.lax.broadcasted_iota(jnp.int32, (_KCAP, nq_cols), 0)
        v_shift = jnp.concatenate([v[:1], v[:_KCAP - 1]], axis=0)
        i_shift = jnp.concatenate([iv[:1], iv[:_KCAP - 1]], axis=0)
        gidx = mi + blk * bk
        newv = jnp.where(krows < pos, v,
                         jnp.where(krows == pos, m, v_shift))
        newi = jnp.where(krows < pos, iv,
                         jnp.where(krows == pos, gidx, i_shift))
        v_ref[...] = jnp.where(upd, newv, v)
        i_ref[...] = jnp.where(upd, newi, iv)
        m2 = jnp.max(s_next, axis=0, keepdims=True)
        cont = jnp.any(m2 > v_ref[_TOPK - 1:_TOPK, :])
        return cont, m2

    jax.lax.while_loop(cond, body, (cont0, m0))

    @pl.when(blk == nblk - 1)
    def _emit():
        vals_ref[...] = v_ref[:_TOPK, :]
        idx_ref[...] = i_ref[:_TOPK, :]


def kernel(queries, keys, k):
    del k
    q, d = queries.shape
    nkeys = keys.shape[0]
    bk = 1024
    nblk = (nkeys + bk - 1) // bk
    kpad = nblk * bk

    norms_text = jnp.linalg.norm(keys, axis=-1)
    norm_question = jnp.linalg.norm(queries, axis=-1)

    qt = queries.T
    keys_pad = jnp.pad(keys, ((0, kpad - nkeys), (0, 0)))
    nk = jnp.pad(norms_text, (0, kpad - nkeys))[:, None]
    nq = norm_question[None, :]

    body = functools.partial(_fused_topk_kernel, nkeys, bk, nblk)
    vals_t, idx_t = pl.pallas_call(
        body,
        grid=(nblk,),
        in_specs=[
            pl.BlockSpec((d, q), lambda i: (0, 0)),
            pl.BlockSpec((1, q), lambda i: (0, 0)),
            pl.BlockSpec((bk, d), lambda i: (i, 0)),
            pl.BlockSpec((bk, 1), lambda i: (i, 0)),
        ],
        out_specs=[
            pl.BlockSpec((_TOPK, q), lambda i: (0, 0)),
            pl.BlockSpec((_TOPK, q), lambda i: (0, 0)),
        ],
        out_shape=[
            jax.ShapeDtypeStruct((_TOPK, q), jnp.float32),
            jax.ShapeDtypeStruct((_TOPK, q), jnp.int32),
        ],
        scratch_shapes=[
            pltpu.VMEM((bk, q), jnp.float32),
            pltpu.VMEM((_KCAP, q), jnp.float32),
            pltpu.VMEM((_KCAP, q), jnp.int32),
        ],
        compiler_params=pltpu.CompilerParams(
            dimension_semantics=("arbitrary",)),
    )(qt, nq, keys_pad, nk)

    return vals_t.T, idx_t.T
